# baseline (device time: 24075 ns/iter reference)
import jax
import jax.numpy as jnp
from jax import lax
from jax.experimental import pallas as pl
from jax.experimental.pallas import tpu as pltpu

N_DEV = 8


def kernel(x, w_mat):
    m, k_per = x.shape
    _, n = w_mat.shape
    m_out = m // N_DEV

    def body(x_ref, w_ref, out_ref, keep_ref, sq_ref, ss_ref, rq_ref,
             rs_ref, qsend_sems, qrecv_sems, ssend_sems, srecv_sems):
        p = lax.axis_index("i")

        barrier_sem = pltpu.get_barrier_semaphore()
        for mask in (1, 3, 4):
            pl.semaphore_signal(
                barrier_sem, inc=1,
                device_id=(jnp.bitwise_xor(p, mask),),
                device_id_type=pl.DeviceIdType.MESH,
            )
        pl.semaphore_wait(barrier_sem, 3)

        def chunk_dot(c):
            return jnp.dot(
                x_ref[pl.ds(c * m_out, m_out), :], w_ref[:, :],
                preferred_element_type=jnp.float32,
            )

        def quantize(val, slot):
            scale = jnp.maximum(jnp.max(jnp.abs(val)), 1e-30) / 127.0
            sq_ref[slot] = jnp.clip(
                jnp.round(val / scale), -127.0, 127.0
            ).astype(jnp.int8)
            ss_ref[slot] = jnp.full((8, 128), scale, jnp.float32)

        def exchange(slot, nslots, partner):
            rdmas = []
            for s in range(slot, slot + nslots):
                data = pltpu.make_async_remote_copy(
                    src_ref=sq_ref.at[s],
                    dst_ref=rq_ref.at[s],
                    send_sem=qsend_sems.at[s],
                    recv_sem=qrecv_sems.at[s],
                    device_id=(partner,),
                    device_id_type=pl.DeviceIdType.MESH,
                )
                sc = pltpu.make_async_remote_copy(
                    src_ref=ss_ref.at[s],
                    dst_ref=rs_ref.at[s],
                    send_sem=ssend_sems.at[s],
                    recv_sem=srecv_sems.at[s],
                    device_id=(partner,),
                    device_id_type=pl.DeviceIdType.MESH,
                )
                data.start()
                sc.start()
                rdmas.append((data, sc))
            return rdmas

        def dequant(slot):
            return rq_ref[slot].astype(jnp.float32) * rs_ref[slot, 0:1, 0:1]

        for i, d in enumerate((1, 2, 5, 6)):
            quantize(chunk_dot(jnp.bitwise_xor(p, d)), i)
        ph1 = exchange(0, 4, jnp.bitwise_xor(p, 1))

        for i, d in enumerate((0, 3, 4, 7)):
            keep_ref[i] = chunk_dot(jnp.bitwise_xor(p, d))

        for i, (data, sc) in enumerate(ph1):
            data.wait_recv()
            sc.wait_recv()
            keep_ref[i] = keep_ref[i] + dequant(i)

        quantize(keep_ref[1], 4)
        quantize(keep_ref[3], 5)
        ph2 = exchange(4, 2, jnp.bitwise_xor(p, 3))
        for (data, sc), (src_slot, keep_slot) in zip(ph2, ((4, 0), (5, 2))):
            data.wait_recv()
            sc.wait_recv()
            keep_ref[keep_slot] = keep_ref[keep_slot] + dequant(src_slot)

        quantize(keep_ref[2], 6)
        ph3 = exchange(6, 1, jnp.bitwise_xor(p, 4))
        ph3[0][0].wait_recv()
        ph3[0][1].wait_recv()

        total = keep_ref[0] + dequant(6)
        out_ref[:, :] = total * jax.nn.sigmoid(total)

        for d, s in ph1 + ph2 + ph3:
            d.wait_send()
            s.wait_send()

    return pl.pallas_call(
        body,
        out_shape=jax.ShapeDtypeStruct((m_out, n), jnp.float32),
        in_specs=[
            pl.BlockSpec(memory_space=pltpu.VMEM),
            pl.BlockSpec(memory_space=pltpu.VMEM),
        ],
        out_specs=pl.BlockSpec(memory_space=pltpu.VMEM),
        scratch_shapes=[
            pltpu.VMEM((4, m_out, n), jnp.float32),
            pltpu.VMEM((7, m_out, n), jnp.int8),
            pltpu.VMEM((7, 8, 128), jnp.float32),
            pltpu.VMEM((7, m_out, n), jnp.int8),
            pltpu.VMEM((7, 8, 128), jnp.float32),
            pltpu.SemaphoreType.DMA((7,)),
            pltpu.SemaphoreType.DMA((7,)),
            pltpu.SemaphoreType.DMA((7,)),
            pltpu.SemaphoreType.DMA((7,)),
        ],
        compiler_params=pltpu.CompilerParams(collective_id=0),
    )(x, w_mat)


# device time: 16057 ns/iter; 1.4993x vs baseline; 1.4993x over previous
import jax
import jax.numpy as jnp
from jax import lax
from jax.experimental import pallas as pl
from jax.experimental.pallas import tpu as pltpu

N_DEV = 8


def kernel(x, w_mat):
    m, k_per = x.shape
    _, n = w_mat.shape
    m_out = m // N_DEV

    def body(x_ref, w_ref, out_ref, sq_ref, ss_ref, rq_ref, rs_ref,
             qsend_sems, qrecv_sems, ssend_sems, srecv_sems):
        p = lax.axis_index("i")

        barrier_sem = pltpu.get_barrier_semaphore()
        for k in range(1, N_DEV):
            peer = lax.rem(p + k, N_DEV)
            pl.semaphore_signal(
                barrier_sem, inc=1,
                device_id=(peer,), device_id_type=pl.DeviceIdType.MESH,
            )

        rdmas = []
        for k in range(1, N_DEV):
            peer = lax.rem(p + k, N_DEV)
            xk = x_ref[pl.ds(peer * m_out, m_out), :]
            c = jnp.dot(xk, w_ref[:, :], preferred_element_type=jnp.float32)
            scale = jnp.maximum(jnp.max(jnp.abs(c)), 1e-30) / 127.0
            sq_ref[k - 1] = jnp.clip(
                jnp.round(c / scale), -127.0, 127.0
            ).astype(jnp.int8)
            ss_ref[k - 1] = jnp.full((8, 128), scale, jnp.float32)

            if k == 1:
                pl.semaphore_wait(barrier_sem, N_DEV - 1)

            data = pltpu.make_async_remote_copy(
                src_ref=sq_ref.at[k - 1],
                dst_ref=rq_ref.at[k - 1],
                send_sem=qsend_sems.at[k - 1],
                recv_sem=qrecv_sems.at[k - 1],
                device_id=(peer,),
                device_id_type=pl.DeviceIdType.MESH,
            )
            data.start()
            sc = pltpu.make_async_remote_copy(
                src_ref=ss_ref.at[k - 1],
                dst_ref=rs_ref.at[k - 1],
                send_sem=ssend_sems.at[k - 1],
                recv_sem=srecv_sems.at[k - 1],
                device_id=(peer,),
                device_id_type=pl.DeviceIdType.MESH,
            )
            sc.start()
            rdmas.append((data, sc))

        total = jnp.dot(
            x_ref[pl.ds(p * m_out, m_out), :], w_ref[:, :],
            preferred_element_type=jnp.float32,
        )

        for k in range(1, N_DEV):
            data, sc = rdmas[k - 1]
            data.wait_recv()
            sc.wait_recv()
            scale = rs_ref[k - 1, 0:1, 0:1]
            total = total + rq_ref[k - 1].astype(jnp.float32) * scale

        out_ref[:, :] = total * jax.nn.sigmoid(total)

        for data, sc in rdmas:
            data.wait_send()
            sc.wait_send()

    return pl.pallas_call(
        body,
        out_shape=jax.ShapeDtypeStruct((m_out, n), jnp.float32),
        in_specs=[
            pl.BlockSpec(memory_space=pltpu.VMEM),
            pl.BlockSpec(memory_space=pltpu.VMEM),
        ],
        out_specs=pl.BlockSpec(memory_space=pltpu.VMEM),
        scratch_shapes=[
            pltpu.VMEM((N_DEV - 1, m_out, n), jnp.int8),
            pltpu.VMEM((N_DEV - 1, 8, 128), jnp.float32),
            pltpu.VMEM((N_DEV - 1, m_out, n), jnp.int8),
            pltpu.VMEM((N_DEV - 1, 8, 128), jnp.float32),
            pltpu.SemaphoreType.DMA((N_DEV - 1,)),
            pltpu.SemaphoreType.DMA((N_DEV - 1,)),
            pltpu.SemaphoreType.DMA((N_DEV - 1,)),
            pltpu.SemaphoreType.DMA((N_DEV - 1,)),
        ],
        compiler_params=pltpu.CompilerParams(collective_id=0),
    )(x, w_mat)
